# Initial kernel scaffold; baseline (speedup 1.0000x reference)
import functools

import jax
import jax.numpy as jnp
from jax.experimental import pallas as pl
from jax.experimental.pallas import tpu as pltpu


def _gat_layer_body(x_ref, adj_ref, w_ref, as_ref, ad_ref, o_ref,
                    wh_ref, wht_ref, fs_ref, fd_ref,
                    *, heads, dh, blk, final):
    i = pl.program_id(0)

    @pl.when(i == 0)
    def _project():
        xv = x_ref[...]
        wv = w_ref[...]
        wh_ref[...] = jax.lax.dot_general(
            xv, wv, (((1,), (0,)), ((), ())),
            preferred_element_type=jnp.float32)
        wht_ref[...] = jax.lax.dot_general(
            wv, xv, (((0,), (1,)), ((), ())),
            preferred_element_type=jnp.float32)
        for h in range(heads):
            sl = slice(h * dh, (h + 1) * dh)
            fs_ref[:, h:h + 1] = jax.lax.dot_general(
                wh_ref[:, sl], as_ref[h:h + 1, :], (((1,), (1,)), ((), ())),
                preferred_element_type=jnp.float32)
            fd_ref[h:h + 1, :] = jax.lax.dot_general(
                ad_ref[h:h + 1, :], wht_ref[sl, :], (((1,), (0,)), ((), ())),
                preferred_element_type=jnp.float32)

    mask = adj_ref[...] > 0.0
    acc = None
    for h in range(heads):
        sl = slice(h * dh, (h + 1) * dh)
        fs_col = fs_ref[pl.ds(i * blk, blk), h:h + 1]
        fd_row = fd_ref[h:h + 1, :]
        z = fs_col + fd_row
        e = jnp.where(z > 0, z, 0.2 * z)
        e = jnp.where(mask, e, jnp.float32(-1e9))
        m = jnp.max(e, axis=1, keepdims=True)
        p = jnp.exp(e - m)
        s = jnp.sum(p, axis=1, keepdims=True)
        alpha = p / s
        out_h = jax.lax.dot_general(
            alpha, wh_ref[:, sl], (((1,), (0,)), ((), ())),
            preferred_element_type=jnp.float32)
        if final:
            acc = out_h if acc is None else acc + out_h
        else:
            o_ref[:, sl] = jnp.where(out_h > 0, out_h, jnp.expm1(out_h))
    if final:
        o_ref[...] = acc * jnp.float32(1.0 / heads)


def _gat_layer(x, adj, w_cat, a_s, a_d, *, final, blk=256):
    n, d_in = x.shape
    heads = a_s.shape[0]
    dh = a_s.shape[1]
    hd = heads * dh
    out_dim = dh if final else hd
    grid = n // blk
    body = functools.partial(_gat_layer_body, heads=heads, dh=dh, blk=blk,
                             final=final)
    return pl.pallas_call(
        body,
        grid=(grid,),
        in_specs=[
            pl.BlockSpec((n, d_in), lambda i: (0, 0)),
            pl.BlockSpec((blk, n), lambda i: (i, 0)),
            pl.BlockSpec((d_in, hd), lambda i: (0, 0)),
            pl.BlockSpec((heads, dh), lambda i: (0, 0)),
            pl.BlockSpec((heads, dh), lambda i: (0, 0)),
        ],
        out_specs=pl.BlockSpec((blk, out_dim), lambda i: (i, 0)),
        out_shape=jax.ShapeDtypeStruct((n, out_dim), jnp.float32),
        scratch_shapes=[
            pltpu.VMEM((n, hd), jnp.float32),
            pltpu.VMEM((hd, n), jnp.float32),
            pltpu.VMEM((n, 8), jnp.float32),
            pltpu.VMEM((8, n), jnp.float32),
        ],
    )(x, adj, w_cat, a_s, a_d)


def kernel(x, adj, Ws, a_src, a_dst, Wf, af_src, af_dst):
    w2 = jnp.transpose(Ws[-1], (1, 0, 2)).reshape(Ws.shape[2], -1)
    y = _gat_layer(x, adj, w2, a_src[-1], a_dst[-1], final=False)
    wf = jnp.transpose(Wf, (1, 0, 2)).reshape(Wf.shape[1], -1)
    return _gat_layer(y, adj, wf, af_src, af_dst, final=True)


# speedup vs baseline: 1.8437x; 1.8437x over previous
import functools

import jax
import jax.numpy as jnp
from jax.experimental import pallas as pl
from jax.experimental.pallas import tpu as pltpu


def _gat_layer_body(x_ref, adj_ref, w_ref, as_ref, ad_ref, o_ref,
                    wh_ref, wht_ref, fs_ref, fd_ref,
                    *, heads, dh, blk, final):
    i = pl.program_id(0)

    @pl.when(i == 0)
    def _project():
        xv = x_ref[...]
        wv = w_ref[...]
        wh_ref[...] = jax.lax.dot_general(
            xv, wv, (((1,), (0,)), ((), ())),
            preferred_element_type=jnp.float32)
        wht_ref[...] = jax.lax.dot_general(
            wv, xv, (((0,), (1,)), ((), ())),
            preferred_element_type=jnp.float32)
        for h in range(heads):
            sl = slice(h * dh, (h + 1) * dh)
            fs_ref[:, h:h + 1] = jax.lax.dot_general(
                wh_ref[:, sl], as_ref[h:h + 1, :], (((1,), (1,)), ((), ())),
                preferred_element_type=jnp.float32)
            fd_ref[h:h + 1, :] = jax.lax.dot_general(
                ad_ref[h:h + 1, :], wht_ref[sl, :], (((1,), (0,)), ((), ())),
                preferred_element_type=jnp.float32)

    mask = adj_ref[...] > 0.0
    acc = None
    for h in range(heads):
        sl = slice(h * dh, (h + 1) * dh)
        fs_col = fs_ref[pl.ds(i * blk, blk), h:h + 1]
        fd_row = fd_ref[h:h + 1, :]
        z = fs_col + fd_row
        e = jnp.where(z > 0, z, 0.2 * z)
        e = jnp.where(mask, e, jnp.float32(-1e9))
        m = jnp.max(e, axis=1, keepdims=True)
        p = jnp.exp(e - m)
        s = jnp.sum(p, axis=1, keepdims=True)
        alpha = p / s
        out_h = jax.lax.dot_general(
            alpha, wh_ref[:, sl], (((1,), (0,)), ((), ())),
            preferred_element_type=jnp.float32)
        if final:
            acc = out_h if acc is None else acc + out_h
        else:
            elu_neg = jnp.exp(jnp.minimum(out_h, 0.0)) - 1.0
            o_ref[:, sl] = jnp.where(out_h > 0, out_h, elu_neg)
    if final:
        o_ref[...] = acc * jnp.float32(1.0 / heads)


def _gat_layer(x, adj, w_cat, a_s, a_d, *, final, blk=256):
    n, d_in = x.shape
    heads = a_s.shape[0]
    dh = a_s.shape[1]
    hd = heads * dh
    out_dim = dh if final else hd
    grid = n // blk
    body = functools.partial(_gat_layer_body, heads=heads, dh=dh, blk=blk,
                             final=final)
    return pl.pallas_call(
        body,
        grid=(grid,),
        in_specs=[
            pl.BlockSpec((n, d_in), lambda i: (0, 0)),
            pl.BlockSpec((blk, n), lambda i: (i, 0)),
            pl.BlockSpec((d_in, hd), lambda i: (0, 0)),
            pl.BlockSpec((heads, dh), lambda i: (0, 0)),
            pl.BlockSpec((heads, dh), lambda i: (0, 0)),
        ],
        out_specs=pl.BlockSpec((blk, out_dim), lambda i: (i, 0)),
        out_shape=jax.ShapeDtypeStruct((n, out_dim), jnp.float32),
        scratch_shapes=[
            pltpu.VMEM((n, hd), jnp.float32),
            pltpu.VMEM((hd, n), jnp.float32),
            pltpu.VMEM((n, 8), jnp.float32),
            pltpu.VMEM((8, n), jnp.float32),
        ],
    )(x, adj, w_cat, a_s, a_d)


def kernel(x, adj, Ws, a_src, a_dst, Wf, af_src, af_dst):
    w2 = jnp.transpose(Ws[-1], (1, 0, 2)).reshape(Ws.shape[2], -1)
    y = _gat_layer(x, adj, w2, a_src[-1], a_dst[-1], final=False)
    wf = jnp.transpose(Wf, (1, 0, 2)).reshape(Wf.shape[1], -1)
    return _gat_layer(y, adj, wf, af_src, af_dst, final=True)


# speedup vs baseline: 2.0163x; 1.0936x over previous
import functools

import jax
import jax.numpy as jnp
from jax.experimental import pallas as pl
from jax.experimental.pallas import tpu as pltpu


def _gat_layer_body(x_ref, adj_ref, w_ref, asbd_ref, adbd_ref, o_ref,
                    wh_ref, wht_ref, fs_ref, fd_ref,
                    *, heads, dh, blk, final):
    i = pl.program_id(0)

    @pl.when(i == 0)
    def _project():
        xv = x_ref[...]
        wv = w_ref[...]
        wh_ref[...] = jax.lax.dot_general(
            xv, wv, (((1,), (0,)), ((), ())),
            preferred_element_type=jnp.float32)
        wht_ref[...] = jax.lax.dot_general(
            wv, xv, (((0,), (1,)), ((), ())),
            preferred_element_type=jnp.float32)
        fs_ref[...] = jax.lax.dot_general(
            wh_ref[...], asbd_ref[...], (((1,), (0,)), ((), ())),
            preferred_element_type=jnp.float32)
        fd_ref[...] = jax.lax.dot_general(
            adbd_ref[...], wht_ref[...], (((1,), (0,)), ((), ())),
            preferred_element_type=jnp.float32)

    mask = adj_ref[...] > 0.0
    acc = None
    for h in range(heads):
        sl = slice(h * dh, (h + 1) * dh)
        fs_col = fs_ref[pl.ds(i * blk, blk), h:h + 1]
        fd_row = fd_ref[h:h + 1, :]
        z = fs_col + fd_row
        e = jnp.maximum(z, 0.2 * z)
        e = jnp.where(mask, e, jnp.float32(-1e9))
        m = jnp.max(e, axis=1, keepdims=True)
        p = jnp.exp(e - m)
        s = jnp.sum(p, axis=1, keepdims=True)
        out_h = jax.lax.dot_general(
            p, wh_ref[:, sl], (((1,), (0,)), ((), ())),
            preferred_element_type=jnp.float32)
        out_h = out_h / s
        if final:
            acc = out_h if acc is None else acc + out_h
        else:
            elu_neg = jnp.exp(jnp.minimum(out_h, 0.0)) - 1.0
            o_ref[:, sl] = jnp.where(out_h > 0, out_h, elu_neg)
    if final:
        o_ref[...] = acc * jnp.float32(1.0 / heads)


def _gat_layer(x, adj, w_cat, a_s_bd, a_d_bd, *, heads, dh, final, blk=256):
    n, d_in = x.shape
    hd = heads * dh
    out_dim = dh if final else hd
    grid = n // blk
    body = functools.partial(_gat_layer_body, heads=heads, dh=dh, blk=blk,
                             final=final)
    return pl.pallas_call(
        body,
        grid=(grid,),
        in_specs=[
            pl.BlockSpec((n, d_in), lambda i: (0, 0)),
            pl.BlockSpec((blk, n), lambda i: (i, 0)),
            pl.BlockSpec((d_in, hd), lambda i: (0, 0)),
            pl.BlockSpec((hd, 8), lambda i: (0, 0)),
            pl.BlockSpec((8, hd), lambda i: (0, 0)),
        ],
        out_specs=pl.BlockSpec((blk, out_dim), lambda i: (i, 0)),
        out_shape=jax.ShapeDtypeStruct((n, out_dim), jnp.float32),
        scratch_shapes=[
            pltpu.VMEM((n, hd), jnp.float32),
            pltpu.VMEM((hd, n), jnp.float32),
            pltpu.VMEM((n, 8), jnp.float32),
            pltpu.VMEM((8, n), jnp.float32),
        ],
    )(x, adj, w_cat, a_s_bd, a_d_bd)


def _block_diag_attn(a, pad=8):
    heads, dh = a.shape
    eye = jnp.eye(heads, pad, dtype=a.dtype)
    return (a[:, :, None] * eye[:, None, :]).reshape(heads * dh, pad)


def kernel(x, adj, Ws, a_src, a_dst, Wf, af_src, af_dst):
    h2, dh2 = a_src.shape[1], a_src.shape[2]
    w2 = jnp.transpose(Ws[-1], (1, 0, 2)).reshape(Ws.shape[2], -1)
    y = _gat_layer(x, adj, w2,
                   _block_diag_attn(a_src[-1]),
                   _block_diag_attn(a_dst[-1]).T,
                   heads=h2, dh=dh2, final=False)
    hf, dhf = af_src.shape
    wf = jnp.transpose(Wf, (1, 0, 2)).reshape(Wf.shape[1], -1)
    return _gat_layer(y, adj, wf,
                      _block_diag_attn(af_src),
                      _block_diag_attn(af_dst).T,
                      heads=hf, dh=dhf, final=True)


# speedup vs baseline: 2.4112x; 1.1959x over previous
import functools
import math

import jax
import jax.numpy as jnp
from jax.experimental import pallas as pl
from jax.experimental.pallas import tpu as pltpu

_LOG2E = math.log2(math.e)


def _gat_layer_body(x_ref, adj_ref, w_ref, asbd_ref, adbd_ref, ones_ref,
                    o_ref, wh_ref, wht_ref, fs_ref, fd_ref, p_ref,
                    *, heads, dh, blk, final):
    i = pl.program_id(0)
    n = wh_ref.shape[0]

    @pl.when(i == 0)
    def _project():
        xv = x_ref[...]
        wv = w_ref[...]
        wh_ref[...] = jax.lax.dot_general(
            xv, wv, (((1,), (0,)), ((), ())),
            preferred_element_type=jnp.float32)
        wht_ref[...] = jax.lax.dot_general(
            wv, xv, (((0,), (1,)), ((), ())),
            preferred_element_type=jnp.float32)
        fs_ref[...] = jax.lax.dot_general(
            wh_ref[...], asbd_ref[...], (((1,), (0,)), ((), ())),
            preferred_element_type=jnp.float32)
        fd_ref[...] = jax.lax.dot_general(
            adbd_ref[...], wht_ref[...], (((1,), (0,)), ((), ())),
            preferred_element_type=jnp.float32)

    adj = adj_ref[...]
    for h in range(heads):
        fs_col = fs_ref[pl.ds(i * blk, blk), h:h + 1]
        fd_row = fd_ref[h:h + 1, :]
        md = jnp.max(fd_row, axis=1, keepdims=True)
        t = fs_col + md
        m = jnp.maximum(t, 0.2 * t)
        z = fs_col + fd_row
        e = jnp.maximum(z, 0.2 * z)
        p_ref[:, h * n:(h + 1) * n] = jnp.exp2(e - m) * adj
    s_all = jax.lax.dot_general(
        p_ref[...], ones_ref[...], (((1,), (0,)), ((), ())),
        preferred_element_type=jnp.float32)
    acc = None
    for h in range(heads):
        sl = slice(h * dh, (h + 1) * dh)
        out_h = jax.lax.dot_general(
            p_ref[:, h * n:(h + 1) * n], wh_ref[:, sl],
            (((1,), (0,)), ((), ())),
            preferred_element_type=jnp.float32)
        out_h = out_h / s_all[:, h:h + 1]
        if final:
            acc = out_h if acc is None else acc + out_h
        else:
            elu_neg = jnp.exp(jnp.minimum(out_h, 0.0)) - 1.0
            o_ref[:, sl] = jnp.where(out_h > 0, out_h, elu_neg)
    if final:
        o_ref[...] = acc * jnp.float32(1.0 / heads)


def _gat_layer(x, adj, w_cat, a_s_bd, a_d_bd, ones_bd, *, heads, dh, final,
               blk=256):
    n, d_in = x.shape
    hd = heads * dh
    out_dim = dh if final else hd
    grid = n // blk
    body = functools.partial(_gat_layer_body, heads=heads, dh=dh, blk=blk,
                             final=final)
    return pl.pallas_call(
        body,
        grid=(grid,),
        in_specs=[
            pl.BlockSpec((n, d_in), lambda i: (0, 0)),
            pl.BlockSpec((blk, n), lambda i: (i, 0)),
            pl.BlockSpec((d_in, hd), lambda i: (0, 0)),
            pl.BlockSpec((hd, 8), lambda i: (0, 0)),
            pl.BlockSpec((8, hd), lambda i: (0, 0)),
            pl.BlockSpec((heads * n, 8), lambda i: (0, 0)),
        ],
        out_specs=pl.BlockSpec((blk, out_dim), lambda i: (i, 0)),
        out_shape=jax.ShapeDtypeStruct((n, out_dim), jnp.float32),
        scratch_shapes=[
            pltpu.VMEM((n, hd), jnp.float32),
            pltpu.VMEM((hd, n), jnp.float32),
            pltpu.VMEM((n, 8), jnp.float32),
            pltpu.VMEM((8, n), jnp.float32),
            pltpu.VMEM((blk, heads * n), jnp.float32),
        ],
    )(x, adj, w_cat, a_s_bd, a_d_bd, ones_bd)


def _block_diag_attn(a, pad=8):
    heads, dh = a.shape
    eye = jnp.eye(heads, pad, dtype=a.dtype)
    return (_LOG2E * a[:, :, None] * eye[:, None, :]).reshape(heads * dh, pad)


def kernel(x, adj, Ws, a_src, a_dst, Wf, af_src, af_dst):
    n = x.shape[0]
    h2, dh2 = a_src.shape[1], a_src.shape[2]
    ones2 = jnp.repeat(jnp.eye(h2, 8, dtype=jnp.float32), n, axis=0)
    w2 = jnp.transpose(Ws[-1], (1, 0, 2)).reshape(Ws.shape[2], -1)
    y = _gat_layer(x, adj, w2,
                   _block_diag_attn(a_src[-1]),
                   _block_diag_attn(a_dst[-1]).T,
                   ones2, heads=h2, dh=dh2, final=False)
    hf, dhf = af_src.shape
    onesf = jnp.repeat(jnp.eye(hf, 8, dtype=jnp.float32), n, axis=0)
    wf = jnp.transpose(Wf, (1, 0, 2)).reshape(Wf.shape[1], -1)
    return _gat_layer(y, adj, wf,
                      _block_diag_attn(af_src),
                      _block_diag_attn(af_dst).T,
                      onesf, heads=hf, dh=dhf, final=True)
